# baseline (speedup 1.0000x reference)
import functools
import math

import jax
import jax.numpy as jnp
from jax import lax
from jax.experimental import pallas as pl
from jax.experimental.pallas import tpu as pltpu
from jax.experimental.pallas import tpu_sc as plsc

N = 1024
V = 100000
PAD = 0
EPS = 0.1
CONF = 1.0 - EPS
U = EPS / (V - 2)
C0 = EPS * math.log(U) + CONF * math.log(CONF)

CW = 1024
K_SC = 24
C_SC = K_SC * CW
CB = 4096
NT = (V - C_SC + CB - 1) // CB
TCW = V - C_SC


NW = 32
RPW = N // NW
L = 16


def _lane_total(a, lane):
    for sh in (8, 4, 2, 1):
        a = a + jnp.take(a, jnp.bitwise_xor(lane, sh))
    return a


@functools.partial(
    pl.kernel,
    mesh=plsc.VectorSubcoreMesh(core_axis_name="c", subcore_axis_name="s"),
    out_type=jax.ShapeDtypeStruct((N,), jnp.float32),
    scratch_types=[
        pltpu.VMEM((RPW,), jnp.int32),
        pltpu.VMEM((RPW, CW), jnp.float32),
        pltpu.VMEM((RPW, L), jnp.float32),
        pltpu.VMEM((RPW, L), jnp.float32),
        pltpu.VMEM((RPW,), jnp.float32),
    ],
)
def _sc_part(preds_hbm, trg_hbm, out_hbm, trg_v, buf, accv, gv, r_v):
    wid = lax.axis_index("s") * 2 + lax.axis_index("c")
    base = wid * RPW
    pltpu.sync_copy(trg_hbm.at[pl.ds(base, RPW)], trg_v)
    zero = jnp.zeros((L,), jnp.float32)
    lane = lax.iota(jnp.int32, L)
    for r in range(RPW):
        accv[r, :] = zero
        gv[r, :] = zero
    t_half = [trg_v[pl.ds(0, L)], trg_v[pl.ds(L, L)]]

    def chunk_body(c, carry):
        cb = c * CW
        pltpu.sync_copy(preds_hbm.at[pl.ds(base, RPW), pl.ds(cb, CW)], buf)
        colbase = jnp.full((L,), cb, jnp.int32) + lane
        for r in range(RPW):
            tb = jnp.take(t_half[r // L], jnp.full((L,), r % L, jnp.int32))
            d = tb - colbase
            acc = accv[r, :]
            g = gv[r, :]
            for k in range(CW // L):
                v = buf[r, pl.ds(k * L, L)]
                acc = acc + v
                g = g + jnp.where(d == k * L, v, 0.0)
            accv[r, :] = acc
            gv[r, :] = g
        return carry

    lax.fori_loop(0, K_SC, chunk_body, 0)

    pltpu.sync_copy(
        preds_hbm.at[pl.ds(base, RPW), pl.ds(0, 128)], buf.at[:, pl.ds(0, 128)]
    )
    idx0 = jnp.zeros((L,), jnp.int32)
    res = [zero, zero]
    for r in range(RPW):
        a = _lane_total(accv[r, :], lane)
        g = _lane_total(gv[r, :], lane)
        p0 = jnp.take(buf[r, pl.ds(0, L)], idx0)
        val = C0 + U * p0 + (U - CONF) * g - U * a
        h = r // L
        res[h] = jnp.where(lane == r % L, val, res[h])
    for h in range(2):
        r_out = jnp.where(t_half[h] != PAD, res[h], 0.0)
        r_v[pl.ds(h * L, L)] = r_out
    pltpu.sync_copy(r_v, out_hbm.at[pl.ds(base, RPW)])


def _tc_body(trg_ref, x_ref, out_ref, acc_ref, gacc_ref):
    j = pl.program_id(0)
    x = x_ref[...]
    trg = trg_ref[...]
    col = jax.lax.broadcasted_iota(jnp.int32, (N, CB), 1) + C_SC + j * CB
    xv = jnp.where(col < V, x, 0.0)
    acc = jnp.sum(xv, axis=1, keepdims=True)
    g = jnp.sum(jnp.where(col == trg, xv, 0.0), axis=1, keepdims=True)

    @pl.when(j == 0)
    def _init():
        acc_ref[...] = acc
        gacc_ref[...] = g

    @pl.when(j > 0)
    def _accum():
        acc_ref[...] += acc
        gacc_ref[...] += g

    @pl.when(j == NT - 1)
    def _final():
        term = (U - CONF) * gacc_ref[...] - U * acc_ref[...]
        out_ref[0, 0] = jnp.sum(jnp.where(trg != PAD, term, 0.0))


def _combine_body(t1_ref, rsc_ref, out_ref):
    out_ref[0, 0] = (t1_ref[0, 0] + jnp.sum(rsc_ref[...])) / N


def kernel(preds, trgs):
    trgs32 = trgs.astype(jnp.int32)
    t1 = pl.pallas_call(
        _tc_body,
        grid=(NT,),
        in_specs=[
            pl.BlockSpec((N, 1), lambda j: (0, 0)),
            pl.BlockSpec((N, CB), lambda j: (0, j + C_SC // CB)),
        ],
        out_specs=pl.BlockSpec((1, 1), lambda j: (0, 0), memory_space=pltpu.SMEM),
        out_shape=jax.ShapeDtypeStruct((1, 1), jnp.float32),
        scratch_shapes=[
            pltpu.VMEM((N, 1), jnp.float32),
            pltpu.VMEM((N, 1), jnp.float32),
        ],
        compiler_params=pltpu.CompilerParams(
            dimension_semantics=("arbitrary",),
        ),
    )(trgs32.reshape(N, 1), preds)
    r_sc = _sc_part(preds, trgs32)
    out = pl.pallas_call(
        _combine_body,
        in_specs=[
            pl.BlockSpec(memory_space=pltpu.SMEM),
            pl.BlockSpec((8, 128), lambda: (0, 0)),
        ],
        out_specs=pl.BlockSpec(memory_space=pltpu.SMEM),
        out_shape=jax.ShapeDtypeStruct((1, 1), jnp.float32),
    )(t1, r_sc.reshape(8, 128))
    return out[0, 0]


# speedup vs baseline: 1.0047x; 1.0047x over previous
import functools
import math

import jax
import jax.numpy as jnp
from jax import lax
from jax.experimental import pallas as pl
from jax.experimental.pallas import tpu as pltpu
from jax.experimental.pallas import tpu_sc as plsc

N = 1024
V = 100000
PAD = 0
EPS = 0.1
CONF = 1.0 - EPS
U = EPS / (V - 2)
C0 = EPS * math.log(U) + CONF * math.log(CONF)

CW = 1024
K_SC = 24
C_SC = K_SC * CW
CB = 4096
NT = (V - C_SC + CB - 1) // CB
TCW = V - C_SC


NW = 32
RPW = N // NW
L = 16


def _lane_total(a, lane):
    for sh in (8, 4, 2, 1):
        a = a + jnp.take(a, jnp.bitwise_xor(lane, sh))
    return a


@functools.partial(
    pl.kernel,
    mesh=plsc.VectorSubcoreMesh(core_axis_name="c", subcore_axis_name="s"),
    out_type=jax.ShapeDtypeStruct((N,), jnp.float32),
    scratch_types=[
        pltpu.VMEM((RPW,), jnp.int32),
        pltpu.VMEM((RPW, CW), jnp.float32),
        pltpu.VMEM((RPW, L), jnp.float32),
        pltpu.VMEM((RPW, L), jnp.float32),
        pltpu.VMEM((RPW,), jnp.float32),
    ],
)
def _sc_part(preds_hbm, trg_hbm, out_hbm, trg_v, buf, accv, gv, r_v):
    wid = lax.axis_index("s") * 2 + lax.axis_index("c")
    base = wid * RPW
    pltpu.sync_copy(trg_hbm.at[pl.ds(base, RPW)], trg_v)
    zero = jnp.zeros((L,), jnp.float32)
    lane = lax.iota(jnp.int32, L)
    for r in range(RPW):
        accv[r, :] = zero
        gv[r, :] = zero
    t_half = [trg_v[pl.ds(0, L)], trg_v[pl.ds(L, L)]]

    def chunk_body(c, carry):
        cb = c * CW
        pltpu.sync_copy(preds_hbm.at[pl.ds(base, RPW), pl.ds(cb, CW)], buf)
        colbase = jnp.full((L,), cb, jnp.int32) + lane
        for r in range(RPW):
            tb = jnp.take(t_half[r // L], jnp.full((L,), r % L, jnp.int32))
            d = tb - colbase
            acc = accv[r, :]
            g = gv[r, :]
            for k in range(CW // L):
                v = buf[r, pl.ds(k * L, L)]
                acc = acc + v
                g = g + jnp.where(d == k * L, v, 0.0)
            accv[r, :] = acc
            gv[r, :] = g
        return carry

    lax.fori_loop(0, K_SC, chunk_body, 0)

    pltpu.sync_copy(
        preds_hbm.at[pl.ds(base, RPW), pl.ds(0, 128)], buf.at[:, pl.ds(0, 128)]
    )
    idx0 = jnp.zeros((L,), jnp.int32)
    res = [zero, zero]
    for r in range(RPW):
        a = _lane_total(accv[r, :], lane)
        g = _lane_total(gv[r, :], lane)
        p0 = jnp.take(buf[r, pl.ds(0, L)], idx0)
        val = C0 + U * p0 + (U - CONF) * g - U * a
        h = r // L
        res[h] = jnp.where(lane == r % L, val, res[h])
    for h in range(2):
        r_out = jnp.where(t_half[h] != PAD, res[h], 0.0)
        r_v[pl.ds(h * L, L)] = r_out
    pltpu.sync_copy(r_v, out_hbm.at[pl.ds(base, RPW)])


def _tc_body(start_blk, nblk, trg_ref, x_ref, out_ref, acc_ref, gacc_ref):
    j = pl.program_id(0)
    x = x_ref[...]
    trg = trg_ref[...]
    col = (
        jax.lax.broadcasted_iota(jnp.int32, (N, CB), 1)
        + start_blk * CB
        + j * CB
    )
    xv = jnp.where(col < V, x, 0.0)
    acc = jnp.sum(xv, axis=1, keepdims=True)
    g = jnp.sum(jnp.where(col == trg, xv, 0.0), axis=1, keepdims=True)

    @pl.when(j == 0)
    def _init():
        acc_ref[...] = acc
        gacc_ref[...] = g

    @pl.when(j > 0)
    def _accum():
        acc_ref[...] += acc
        gacc_ref[...] += g

    @pl.when(j == nblk - 1)
    def _final():
        term = (U - CONF) * gacc_ref[...] - U * acc_ref[...]
        out_ref[0, 0] = jnp.sum(jnp.where(trg != PAD, term, 0.0))


def _tc_part(preds, trg2, start_blk, nblk):
    return pl.pallas_call(
        functools.partial(_tc_body, start_blk, nblk),
        grid=(nblk,),
        in_specs=[
            pl.BlockSpec((N, 1), lambda j: (0, 0)),
            pl.BlockSpec((N, CB), lambda j: (0, j + start_blk)),
        ],
        out_specs=pl.BlockSpec((1, 1), lambda j: (0, 0), memory_space=pltpu.SMEM),
        out_shape=jax.ShapeDtypeStruct((1, 1), jnp.float32),
        scratch_shapes=[
            pltpu.VMEM((N, 1), jnp.float32),
            pltpu.VMEM((N, 1), jnp.float32),
        ],
        compiler_params=pltpu.CompilerParams(
            dimension_semantics=("arbitrary",),
        ),
    )(trg2, preds)


def _combine_body(t1a_ref, t1b_ref, rsc_ref, out_ref):
    out_ref[0, 0] = (
        t1a_ref[0, 0] + t1b_ref[0, 0] + jnp.sum(rsc_ref[...])
    ) / N


NT_A = 9
NT_B = NT - NT_A


def kernel(preds, trgs):
    trgs32 = trgs.astype(jnp.int32)
    trg2 = trgs32.reshape(N, 1)
    t1a = _tc_part(preds, trg2, C_SC // CB, NT_A)
    r_sc = _sc_part(preds, trgs32)
    t1b = _tc_part(preds, trg2, C_SC // CB + NT_A, NT_B)
    out = pl.pallas_call(
        _combine_body,
        in_specs=[
            pl.BlockSpec(memory_space=pltpu.SMEM),
            pl.BlockSpec(memory_space=pltpu.SMEM),
            pl.BlockSpec((8, 128), lambda: (0, 0)),
        ],
        out_specs=pl.BlockSpec(memory_space=pltpu.SMEM),
        out_shape=jax.ShapeDtypeStruct((1, 1), jnp.float32),
    )(t1a, t1b, r_sc.reshape(8, 128))
    return out[0, 0]


# speedup vs baseline: 1.0926x; 1.0874x over previous
import functools
import math

import jax
import jax.numpy as jnp
from jax import lax
from jax.experimental import pallas as pl
from jax.experimental.pallas import tpu as pltpu
from jax.experimental.pallas import tpu_sc as plsc

N = 1024
V = 100000
PAD = 0
EPS = 0.1
CONF = 1.0 - EPS
U = EPS / (V - 2)
C0 = EPS * math.log(U) + CONF * math.log(CONF)

CW = 512
K_SC = 48
C_SC = K_SC * CW
CB = 4096
NT = (V - C_SC + CB - 1) // CB
TCW = V - C_SC


NW = 32
RPW = N // NW
L = 16


def _lane_total(a, lane):
    for sh in (8, 4, 2, 1):
        a = a + jnp.take(a, jnp.bitwise_xor(lane, sh))
    return a


@functools.partial(
    pl.kernel,
    mesh=plsc.VectorSubcoreMesh(core_axis_name="c", subcore_axis_name="s"),
    out_type=jax.ShapeDtypeStruct((N,), jnp.float32),
    scratch_types=[
        pltpu.VMEM((RPW,), jnp.int32),
        pltpu.VMEM((RPW, CW), jnp.float32),
        pltpu.VMEM((RPW, CW), jnp.float32),
        pltpu.VMEM((RPW, L), jnp.float32),
        pltpu.VMEM((RPW, L), jnp.float32),
        pltpu.VMEM((RPW,), jnp.float32),
        pltpu.SemaphoreType.DMA,
        pltpu.SemaphoreType.DMA,
    ],
)
def _sc_part(preds_hbm, trg_hbm, out_hbm, trg_v, buf0, buf1, accv, gv, r_v,
             sem0, sem1):
    wid = lax.axis_index("s") * 2 + lax.axis_index("c")
    base = wid * RPW
    pltpu.sync_copy(trg_hbm.at[pl.ds(base, RPW)], trg_v)
    zero = jnp.zeros((L,), jnp.float32)
    lane = lax.iota(jnp.int32, L)
    for r in range(RPW):
        accv[r, :] = zero
        gv[r, :] = zero
    t_half = [trg_v[pl.ds(0, L)], trg_v[pl.ds(L, L)]]

    def src(cb):
        return preds_hbm.at[pl.ds(base, RPW), pl.ds(cb, CW)]

    def process(buf, cb):
        colbase = jnp.full((L,), cb, jnp.int32) + lane
        for r in range(RPW):
            tb = jnp.take(t_half[r // L], jnp.full((L,), r % L, jnp.int32))
            d = tb - colbase
            acc = accv[r, :]
            g = gv[r, :]
            for k in range(CW // L):
                v = buf[r, pl.ds(k * L, L)]
                acc = acc + v
                g = g + jnp.where(d == k * L, v, 0.0)
            accv[r, :] = acc
            gv[r, :] = g

    pltpu.make_async_copy(src(0), buf0, sem0).start()
    pltpu.make_async_copy(src(CW), buf1, sem1).start()

    def pair_body(i, carry):
        for b, buf, sem in ((0, buf0, sem0), (1, buf1, sem1)):
            c = 2 * i + b
            cb = c * CW
            pltpu.make_async_copy(src(cb), buf, sem).wait()
            process(buf, cb)
            nxt = cb + 2 * CW

            @pl.when(c + 2 < K_SC)
            def _prefetch():
                pltpu.make_async_copy(src(nxt), buf, sem).start()

        return carry

    lax.fori_loop(0, K_SC // 2, pair_body, 0)

    pltpu.sync_copy(
        preds_hbm.at[pl.ds(base, RPW), pl.ds(0, 128)], buf0.at[:, pl.ds(0, 128)]
    )
    idx0 = jnp.zeros((L,), jnp.int32)
    res = [zero, zero]
    for r in range(RPW):
        a = _lane_total(accv[r, :], lane)
        g = _lane_total(gv[r, :], lane)
        p0 = jnp.take(buf0[r, pl.ds(0, L)], idx0)
        val = C0 + U * p0 + (U - CONF) * g - U * a
        h = r // L
        res[h] = jnp.where(lane == r % L, val, res[h])
    for h in range(2):
        r_out = jnp.where(t_half[h] != PAD, res[h], 0.0)
        r_v[pl.ds(h * L, L)] = r_out
    pltpu.sync_copy(r_v, out_hbm.at[pl.ds(base, RPW)])


def _tc_body(start_blk, nblk, trg_ref, x_ref, out_ref, acc_ref, gacc_ref):
    j = pl.program_id(0)
    x = x_ref[...]
    trg = trg_ref[...]
    col = (
        jax.lax.broadcasted_iota(jnp.int32, (N, CB), 1)
        + start_blk * CB
        + j * CB
    )
    xv = jnp.where(col < V, x, 0.0)
    acc = jnp.sum(xv, axis=1, keepdims=True)
    g = jnp.sum(jnp.where(col == trg, xv, 0.0), axis=1, keepdims=True)

    @pl.when(j == 0)
    def _init():
        acc_ref[...] = acc
        gacc_ref[...] = g

    @pl.when(j > 0)
    def _accum():
        acc_ref[...] += acc
        gacc_ref[...] += g

    @pl.when(j == nblk - 1)
    def _final():
        term = (U - CONF) * gacc_ref[...] - U * acc_ref[...]
        out_ref[0, 0] = jnp.sum(jnp.where(trg != PAD, term, 0.0))


def _tc_part(preds, trg2, start_blk, nblk):
    return pl.pallas_call(
        functools.partial(_tc_body, start_blk, nblk),
        grid=(nblk,),
        in_specs=[
            pl.BlockSpec((N, 1), lambda j: (0, 0)),
            pl.BlockSpec((N, CB), lambda j: (0, j + start_blk)),
        ],
        out_specs=pl.BlockSpec((1, 1), lambda j: (0, 0), memory_space=pltpu.SMEM),
        out_shape=jax.ShapeDtypeStruct((1, 1), jnp.float32),
        scratch_shapes=[
            pltpu.VMEM((N, 1), jnp.float32),
            pltpu.VMEM((N, 1), jnp.float32),
        ],
        compiler_params=pltpu.CompilerParams(
            dimension_semantics=("arbitrary",),
        ),
    )(trg2, preds)


def _combine_body(t1a_ref, t1b_ref, rsc_ref, out_ref):
    out_ref[0, 0] = (
        t1a_ref[0, 0] + t1b_ref[0, 0] + jnp.sum(rsc_ref[...])
    ) / N


NT_A = 9
NT_B = NT - NT_A


def kernel(preds, trgs):
    trgs32 = trgs.astype(jnp.int32)
    trg2 = trgs32.reshape(N, 1)
    t1a = _tc_part(preds, trg2, C_SC // CB, NT_A)
    r_sc = _sc_part(preds, trgs32)
    t1b = _tc_part(preds, trg2, C_SC // CB + NT_A, NT_B)
    out = pl.pallas_call(
        _combine_body,
        in_specs=[
            pl.BlockSpec(memory_space=pltpu.SMEM),
            pl.BlockSpec(memory_space=pltpu.SMEM),
            pl.BlockSpec((8, 128), lambda: (0, 0)),
        ],
        out_specs=pl.BlockSpec(memory_space=pltpu.SMEM),
        out_shape=jax.ShapeDtypeStruct((1, 1), jnp.float32),
    )(t1a, t1b, r_sc.reshape(8, 128))
    return out[0, 0]
